# baseline (speedup 1.0000x reference)
import dataclasses

import jax
import jax.numpy as jnp
from jax import lax
from jax.experimental import pallas as pl
from jax.experimental.pallas import tpu as pltpu
from jax.experimental.pallas import tpu_sc as plsc

NUM_POS = 128
NUM_NEG = 128
LANES = 128


def _match_topk_body(n_tgt, n_rows, boxes_ref, tgt_ref, cls_ref,
                     sel_ref, keep_ref, cls_out_ref):
    x0 = boxes_ref[0]
    y0 = boxes_ref[1]
    x1 = boxes_ref[2]
    y1 = boxes_ref[3]
    area1 = (x1 - x0) * (y1 - y0)

    init_max = jnp.full((n_rows, LANES), -jnp.inf, jnp.float32)
    init_arg = jnp.zeros((n_rows, LANES), jnp.int32)

    def tloop(j, carry):
        cmax, carg = carry
        tx0 = tgt_ref[0, j]
        ty0 = tgt_ref[1, j]
        tx1 = tgt_ref[2, j]
        ty1 = tgt_ref[3, j]
        area2 = (tx1 - tx0) * (ty1 - ty0)
        ltx = jnp.maximum(x0, tx0)
        lty = jnp.maximum(y0, ty0)
        rbx = jnp.minimum(x1, tx1)
        rby = jnp.minimum(y1, ty1)
        w = jnp.maximum(rbx - ltx, 0.0)
        h = jnp.maximum(rby - lty, 0.0)
        inter = w * h
        union = (area1 + area2) - inter
        iou = inter / jnp.maximum(union, 1e-6)
        better = iou > cmax
        cmax = jnp.where(better, iou, cmax)
        carg = jnp.where(better, j, carg)
        return cmax, carg

    cmax, carg = lax.fori_loop(0, n_tgt, tloop, (init_max, init_arg))

    lin = (lax.broadcasted_iota(jnp.int32, (n_rows, LANES), 0) * LANES
           + lax.broadcasted_iota(jnp.int32, (n_rows, LANES), 1))

    def kloop(k, scores):
        m = jnp.max(scores)
        idx = jnp.min(jnp.where(scores == m, lin, jnp.int32(2**30)))
        hit = lin == idx
        mt = jnp.max(jnp.where(hit, carg, -1))
        valid = m >= 0.7
        sel_ref[k] = jnp.where(valid, idx, 0)
        keep_ref[k] = jnp.where(valid, jnp.float32(1.0), jnp.float32(0.0))
        cls_out_ref[k] = jnp.where(valid, cls_ref[mt], -1)
        return jnp.where(hit, jnp.float32(-3.0), scores)

    lax.fori_loop(0, NUM_POS, kloop, cmax)

    def nloop(k, c):
        cls_out_ref[k] = -1
        return c

    lax.fori_loop(NUM_POS, NUM_POS + NUM_NEG, nloop, 0)


def _match_topk(boxes3, tgt_t, classes):
    n_rows = boxes3.shape[1]
    n_tgt = tgt_t.shape[1]
    return pl.pallas_call(
        lambda *refs: _match_topk_body(n_tgt, n_rows, *refs),
        in_specs=[
            pl.BlockSpec(memory_space=pltpu.VMEM),
            pl.BlockSpec(memory_space=pltpu.SMEM),
            pl.BlockSpec(memory_space=pltpu.SMEM),
        ],
        out_specs=[
            pl.BlockSpec(memory_space=pltpu.SMEM),
            pl.BlockSpec(memory_space=pltpu.SMEM),
            pl.BlockSpec(memory_space=pltpu.SMEM),
        ],
        out_shape=[
            jax.ShapeDtypeStruct((NUM_POS,), jnp.int32),
            jax.ShapeDtypeStruct((NUM_POS,), jnp.float32),
            jax.ShapeDtypeStruct((NUM_POS + NUM_NEG,), classes.dtype),
        ],
    )(boxes3, tgt_t, classes)


def _sc_gather(embeddings, sel_idx, keep):
    d_emb = embeddings.shape[1]
    n_out = NUM_POS + NUM_NEG
    rows_per_w = 8
    n_gather_w = NUM_POS // rows_per_w
    mesh = plsc.VectorSubcoreMesh(core_axis_name="c", subcore_axis_name="s")
    nc = 2

    def body(emb_hbm, idx_hbm, keep_hbm, out_hbm, idx_v, keep_v, rows_v, sem):
        wid = lax.axis_index("s") * nc + lax.axis_index("c")
        base = wid * rows_per_w

        @pl.when(wid < n_gather_w)
        def _():
            pltpu.sync_copy(idx_hbm.at[pl.ds(base, rows_per_w)], idx_v)
            pltpu.sync_copy(keep_hbm, keep_v)
            pltpu.async_copy(emb_hbm.at[idx_v], rows_v, sem).wait()
            for r in range(rows_per_w):
                kvec = plsc.load_gather(
                    keep_v, [jnp.full((16,), base + r, jnp.int32)])
                for c in range(d_emb // 16):
                    sl = (r, pl.ds(c * 16, 16))
                    rows_v[sl] = rows_v[sl] * kvec
            pltpu.sync_copy(rows_v, out_hbm.at[pl.ds(base, rows_per_w)])

        @pl.when(wid >= n_gather_w)
        def _():
            zero = jnp.zeros((16,), jnp.float32)
            for r in range(rows_per_w):
                for c in range(d_emb // 16):
                    rows_v[r, pl.ds(c * 16, 16)] = zero
            pltpu.sync_copy(rows_v, out_hbm.at[pl.ds(base, rows_per_w)])

    cp = pltpu.CompilerParams()
    if "needs_layout_passes" in pltpu.CompilerParams.__dataclass_fields__:
        cp = dataclasses.replace(cp, needs_layout_passes=False)
    return pl.kernel(
        body,
        out_type=jax.ShapeDtypeStruct((n_out, d_emb), jnp.float32),
        mesh=mesh,
        compiler_params=cp,
        scratch_types=[
            pltpu.VMEM((rows_per_w,), jnp.int32),
            pltpu.VMEM((NUM_POS,), jnp.float32),
            pltpu.VMEM((rows_per_w, d_emb), jnp.float32),
            pltpu.SemaphoreType.DMA,
        ],
    )(embeddings, sel_idx, keep)


def kernel(embeddings, proposals, targets_boxes, target_classes):
    n_all = proposals.shape[0] + targets_boxes.shape[0]
    n_rows = (n_all + LANES - 1) // LANES
    pad = n_rows * LANES - n_all
    all_boxes = jnp.concatenate(
        [proposals, targets_boxes,
         jnp.zeros((pad, 4), jnp.float32)], axis=0)
    boxes3 = all_boxes.T.reshape(4, n_rows, LANES)
    tgt_t = targets_boxes.T
    sel_idx, keep, sampled_cls = _match_topk(boxes3, tgt_t, target_classes)
    out_emb = _sc_gather(embeddings, sel_idx, keep)
    return out_emb, sampled_cls
